# Initial kernel scaffold; baseline (speedup 1.0000x reference)
import jax
import jax.numpy as jnp
from jax.experimental import pallas as pl


def kernel(Gu, Gi, Gut, Git, edge_index):
    raise NotImplementedError("write your pallas kernel here")


# speedup vs baseline: 10.0026x; 10.0026x over previous
import functools

import jax
import jax.numpy as jnp
from jax import lax
from jax.experimental import pallas as pl
from jax.experimental.pallas import tpu as pltpu
from jax.experimental.pallas import tpu_sc as plsc

NUM_USERS = 70000
NUM_ITEMS = 30000
N_NODES = NUM_USERS + NUM_ITEMS
N_EDGES = 1600000
KH = 16
N_TILES = 16
RPT = 6272
NPAD = N_TILES * RPT
EPT = 102400
EPAD = N_TILES * EPT
EB = 128
RB = 224
N_LAYERS = 3


def _fill_rows(ref, n_rows, vec):
    for r in range(n_rows):
        ref[r] = vec


def _rsqrt16(x):
    i = lax.bitcast_convert_type(x, jnp.int32)
    i = jnp.int32(0x5F3759DF) - lax.shift_right_logical(i, 1)
    y = lax.bitcast_convert_type(i, jnp.float32)
    for _ in range(3):
        y = y * (1.5 - 0.5 * x * y * y)
    return y


def _sc_body(y_hbm, src_hbm, dst_hbm, out_hbm, xs_hbm,
             acc_sh, deg_sh, src_v, dst_v, rows_v, one_v, zero_v, zrows,
             disv, accb, outb, sem, *, tile):
    row0 = tile * RPT
    ebase = tile * EPT

    def zero_deg(j, c):
        pltpu.sync_copy(zero_v, deg_sh.at[pl.ds(row0 + j * EB, EB)])
        return c
    lax.fori_loop(0, RPT // EB, zero_deg, 0)
    plsc.subcore_barrier()

    def deg_step(e, c):
        pltpu.sync_copy(dst_hbm.at[pl.ds(ebase + e * EB, EB)], dst_v)
        pltpu.sync_copy(one_v, deg_sh.at[dst_v], add=True)
        return c
    lax.fori_loop(0, EPT // EB, deg_step, 0)
    plsc.subcore_barrier()

    pltpu.sync_copy(deg_sh.at[pl.ds(row0, RPT)], disv)

    def dis_step(i, c):
        d16 = disv[pl.ds(i * 16, 16)]
        y = _rsqrt16(jnp.maximum(d16, 1.0))
        disv[pl.ds(i * 16, 16)] = jnp.where(d16 > 0.0, y, 0.0)
        return c
    lax.fori_loop(0, RPT // 16, dis_step, 0)

    def p0_chunk(j, c):
        r = row0 + j * RB
        pltpu.sync_copy(y_hbm.at[pl.ds(r, RB), :], accb)

        def p0_grp(g, c2):
            d16 = disv[pl.ds(j * RB + g * 16, 16)]
            for q in range(16):
                i = g * 16 + q
                a = accb[i]
                outb[i] = 0.25 * a
                accb[i] = d16[q] * a
            return c2
        lax.fori_loop(0, RB // 16, p0_grp, 0)
        pltpu.sync_copy(outb, out_hbm.at[pl.ds(r, RB), :])
        pltpu.sync_copy(accb, xs_hbm.at[pl.ds(r, RB), :])
        return c
    lax.fori_loop(0, RPT // RB, p0_chunk, 0)

    for _l in range(N_LAYERS):
        def zero_acc(j, c):
            pltpu.sync_copy(zrows, acc_sh.at[pl.ds(row0 + j * EB, EB), :])
            return c
        lax.fori_loop(0, RPT // EB, zero_acc, 0)
        plsc.subcore_barrier()

        def edge_step(e, c):
            b = ebase + e * EB
            pltpu.sync_copy(src_hbm.at[pl.ds(b, EB)], src_v)
            pltpu.sync_copy(dst_hbm.at[pl.ds(b, EB)], dst_v)
            pltpu.async_copy(xs_hbm.at[src_v], rows_v, sem).wait()
            pltpu.sync_copy(rows_v, acc_sh.at[dst_v], add=True)
            return c
        lax.fori_loop(0, EPT // EB, edge_step, 0)
        plsc.subcore_barrier()

        def sc_chunk(j, c):
            r = row0 + j * RB
            pltpu.sync_copy(acc_sh.at[pl.ds(r, RB), :], accb)
            pltpu.sync_copy(out_hbm.at[pl.ds(r, RB), :], outb)

            def sc_grp(g, c2):
                d16 = disv[pl.ds(j * RB + g * 16, 16)]
                for q in range(16):
                    i = g * 16 + q
                    a = accb[i]
                    d = d16[q]
                    outb[i] = outb[i] + (0.25 * d) * a
                    accb[i] = (d * d) * a
                return c2
            lax.fori_loop(0, RB // 16, sc_grp, 0)
            pltpu.sync_copy(outb, out_hbm.at[pl.ds(r, RB), :])
            pltpu.sync_copy(accb, xs_hbm.at[pl.ds(r, RB), :])
            return c
        lax.fori_loop(0, RPT // RB, sc_chunk, 0)
        plsc.subcore_barrier()


def _make_kernel():
    mesh = plsc.VectorSubcoreMesh(core_axis_name="c", subcore_axis_name="s")

    @functools.partial(
        pl.kernel,
        mesh=mesh,
        out_type=[
            jax.ShapeDtypeStruct((NPAD, KH), jnp.float32),
            jax.ShapeDtypeStruct((NPAD, KH), jnp.float32),
            jax.ShapeDtypeStruct((NPAD, KH), jnp.float32),
            jax.ShapeDtypeStruct((NPAD, KH), jnp.float32),
        ],
        scratch_types=[
            pltpu.VMEM_SHARED((NPAD, KH), jnp.float32),
            pltpu.VMEM_SHARED((NPAD,), jnp.float32),
            pltpu.VMEM((EB,), jnp.int32),
            pltpu.VMEM((EB,), jnp.int32),
            pltpu.VMEM((EB, KH), jnp.float32),
            pltpu.VMEM((EB,), jnp.float32),
            pltpu.VMEM((EB,), jnp.float32),
            pltpu.VMEM((EB, KH), jnp.float32),
            pltpu.VMEM((RPT,), jnp.float32),
            pltpu.VMEM((RB, KH), jnp.float32),
            pltpu.VMEM((RB, KH), jnp.float32),
            pltpu.SemaphoreType.DMA,
        ],
        compiler_params=pltpu.CompilerParams(use_tc_tiling_on_sc=False),
    )
    def _k(yL, yR, src_hbm, dst_hbm, outL, outR, xsL, xsR,
           acc_sh, deg_sh, src_v, dst_v, rows_v, one_v, zero_v, zrows,
           disv, accb, outb, sem):
        core = lax.axis_index("c")
        tile = lax.axis_index("s")

        zv = jnp.zeros((16,), jnp.float32)
        _fill_rows(zrows, EB, zv)
        ov = jnp.ones((16,), jnp.float32)
        for r in range(EB // 16):
            one_v[pl.ds(r * 16, 16)] = ov
            zero_v[pl.ds(r * 16, 16)] = zv

        scratch = (acc_sh, deg_sh, src_v, dst_v, rows_v, one_v, zero_v,
                   zrows, disv, accb, outb, sem)

        @pl.when(core == 0)
        def _():
            _sc_body(yL, src_hbm, dst_hbm, outL, xsL, *scratch, tile=tile)

        @pl.when(core == 1)
        def _():
            _sc_body(yR, src_hbm, dst_hbm, outR, xsR, *scratch, tile=tile)

    return _k


_SC_KERNEL = _make_kernel()


def kernel(Gu, Gi, Gut, Git, edge_index):
    y0 = jnp.concatenate([Gu + Gut, Gi + Git], axis=0)
    y0 = jnp.pad(y0, ((0, NPAD - N_NODES), (0, 0)))
    yL = y0[:, :KH]
    yR = y0[:, KH:]
    fill = jnp.full((EPAD - N_EDGES,), NPAD - 1, dtype=jnp.int32)
    src = jnp.concatenate([edge_index[0], fill])
    dst = jnp.concatenate([edge_index[1], fill])
    outL, outR, _, _ = _SC_KERNEL(yL, yR, src, dst)
    return jnp.concatenate([outL[:N_NODES], outR[:N_NODES]], axis=1)


# speedup vs baseline: 13.6009x; 1.3597x over previous
import functools

import jax
import jax.numpy as jnp
from jax import lax
from jax.experimental import pallas as pl
from jax.experimental.pallas import tpu as pltpu
from jax.experimental.pallas import tpu_sc as plsc

NUM_USERS = 70000
NUM_ITEMS = 30000
N_NODES = NUM_USERS + NUM_ITEMS
N_EDGES = 1600000
KH = 16
N_TILES = 16
RPT = 6272
NPAD = N_TILES * RPT
EPT = 102400
EPAD = N_TILES * EPT
EB = 128
MB = 4
NG = EPT // (MB * EB)
RB = 128
N_LAYERS = 3
ROW_BYTES = EB * KH * 4


def _rsqrt16(x):
    i = lax.bitcast_convert_type(x, jnp.int32)
    i = jnp.int32(0x5F3759DF) - lax.shift_right_logical(i, 1)
    y = lax.bitcast_convert_type(i, jnp.float32)
    for _ in range(3):
        y = y * (1.5 - 0.5 * x * y * y)
    return y


def _sc_body(y_hbm, src2, dst1, e1h, zh, out_hbm, xs_hbm, dis_hbm,
             acc_sh, srcb, dsts, rows, disb, accb, outb, semg, sems,
             *, tile):
    row0 = tile * RPT
    et0 = tile * (EPT // EB)

    def load_dst(g, slot):
        e0 = (et0 + g * MB) * EB
        for j in range(MB):
            pltpu.sync_copy(dst1.at[pl.ds(e0 + j * EB, EB)],
                            dsts[slot * MB + j])

    def load_src(g, slot):
        r = et0 + g * MB
        pltpu.sync_copy(src2.at[pl.ds(r, MB), :],
                        srcb.at[pl.ds(slot * MB, MB), :])

    pltpu.sync_copy(zh, acc_sh.at[pl.ds(row0, RPT), :])
    plsc.subcore_barrier()

    pltpu.sync_copy(e1h, rows.at[0])

    def deg_body(g, c):
        load_dst(g, 0)
        hs = [pltpu.async_copy(rows.at[0], acc_sh.at[dsts[j]], sems,
                               add=True) for j in range(MB)]
        for h in hs:
            h.wait()
        return c
    lax.fori_loop(0, NG, deg_body, 0)
    plsc.subcore_barrier()

    def dis_chunk(j, c):
        r = row0 + j * RB
        pltpu.sync_copy(acc_sh.at[pl.ds(r, RB), :], accb)

        def dis_row(i, c2):
            drow = accb[i]
            y = _rsqrt16(jnp.maximum(drow, 1.0))
            accb[i] = jnp.where(drow > 0.0, y, 0.0)
            return c2
        lax.fori_loop(0, RB, dis_row, 0)
        pltpu.sync_copy(accb, dis_hbm.at[pl.ds(r, RB), :])
        return c
    lax.fori_loop(0, RPT // RB, dis_chunk, 0)

    pltpu.sync_copy(zh, acc_sh.at[pl.ds(row0, RPT), :])

    def p0_chunk(j, c):
        r = row0 + j * RB
        pltpu.sync_copy(y_hbm.at[pl.ds(r, RB), :], accb)
        pltpu.sync_copy(dis_hbm.at[pl.ds(r, RB), :], disb)

        def p0_row(i, c2):
            a = accb[i]
            outb[i] = 0.25 * a
            accb[i] = disb[i] * a
            return c2
        lax.fori_loop(0, RB, p0_row, 0)
        pltpu.sync_copy(outb, out_hbm.at[pl.ds(r, RB), :])
        pltpu.sync_copy(accb, xs_hbm.at[pl.ds(r, RB), :])
        return c
    lax.fori_loop(0, RPT // RB, p0_chunk, 0)
    plsc.subcore_barrier()

    for l in range(N_LAYERS):
        def edge_body(g, c):
            load_src(g, 0)
            load_dst(g, 0)
            hg = [pltpu.async_copy(xs_hbm.at[srcb.at[j]], rows.at[j], semg)
                  for j in range(MB)]
            for h in hg:
                h.wait()
            hs = [pltpu.async_copy(rows.at[j], acc_sh.at[dsts[j]], sems,
                                   add=True) for j in range(MB)]
            for h in hs:
                h.wait()
            return c
        lax.fori_loop(0, NG, edge_body, 0)
        plsc.subcore_barrier()

        last = l == N_LAYERS - 1

        def sc_chunk(j, c):
            r = row0 + j * RB
            pltpu.sync_copy(acc_sh.at[pl.ds(r, RB), :], accb)
            pltpu.sync_copy(out_hbm.at[pl.ds(r, RB), :], outb)
            pltpu.sync_copy(dis_hbm.at[pl.ds(r, RB), :], disb)

            def sc_row(i, c2):
                a = accb[i]
                d = disb[i]
                outb[i] = outb[i] + (0.25 * d) * a
                if not last:
                    accb[i] = (d * d) * a
                return c2
            lax.fori_loop(0, RB, sc_row, 0)
            pltpu.sync_copy(outb, out_hbm.at[pl.ds(r, RB), :])
            if not last:
                pltpu.sync_copy(accb, xs_hbm.at[pl.ds(r, RB), :])
            return c
        lax.fori_loop(0, RPT // RB, sc_chunk, 0)
        if not last:
            pltpu.sync_copy(zh, acc_sh.at[pl.ds(row0, RPT), :])
            plsc.subcore_barrier()


def _make_kernel():
    mesh = plsc.VectorSubcoreMesh(core_axis_name="c", subcore_axis_name="s")

    @functools.partial(
        pl.kernel,
        mesh=mesh,
        out_type=[
            jax.ShapeDtypeStruct((NPAD, KH), jnp.float32),
            jax.ShapeDtypeStruct((NPAD, KH), jnp.float32),
            jax.ShapeDtypeStruct((NPAD, KH), jnp.float32),
            jax.ShapeDtypeStruct((NPAD, KH), jnp.float32),
            jax.ShapeDtypeStruct((NPAD, KH), jnp.float32),
            jax.ShapeDtypeStruct((NPAD, KH), jnp.float32),
        ],
        scratch_types=[
            pltpu.VMEM_SHARED((NPAD, KH), jnp.float32),
            pltpu.VMEM((MB, EB), jnp.int32),
            [pltpu.VMEM((EB,), jnp.int32)] * MB,
            pltpu.VMEM((MB, EB, KH), jnp.float32),
            pltpu.VMEM((RB, KH), jnp.float32),
            pltpu.VMEM((RB, KH), jnp.float32),
            pltpu.VMEM((RB, KH), jnp.float32),
            pltpu.SemaphoreType.DMA,
            pltpu.SemaphoreType.DMA,
        ],
        compiler_params=pltpu.CompilerParams(use_tc_tiling_on_sc=False),
    )
    def _k(yL, yR, src2, dst1, e1h, zh, outL, outR, xsL, xsR, disL, disR,
           acc_sh, srcb, dsts, rows, disb, accb, outb, semg, sems):
        core = lax.axis_index("c")
        tile = lax.axis_index("s")
        scratch = (acc_sh, srcb, dsts, rows, disb, accb, outb, semg, sems)

        @pl.when(core == 0)
        def _():
            _sc_body(yL, src2, dst1, e1h, zh, outL, xsL, disL, *scratch,
                     tile=tile)

        @pl.when(core == 1)
        def _():
            _sc_body(yR, src2, dst1, e1h, zh, outR, xsR, disR, *scratch,
                     tile=tile)

    return _k


_SC_KERNEL = _make_kernel()


def kernel(Gu, Gi, Gut, Git, edge_index):
    y0 = jnp.concatenate([Gu + Gut, Gi + Git], axis=0)
    y0 = jnp.pad(y0, ((0, NPAD - N_NODES), (0, 0)))
    yL = y0[:, :KH]
    yR = y0[:, KH:]
    fill = jnp.full((EPAD - N_EDGES,), NPAD - 1, dtype=jnp.int32)
    src = jnp.concatenate([edge_index[0], fill]).reshape(EPAD // EB, EB)
    dst = jnp.concatenate([edge_index[1], fill])
    e1h = jnp.ones((EB, KH), jnp.float32)
    zh = jnp.zeros((RPT, KH), jnp.float32)
    outL, outR, _, _, _, _ = _SC_KERNEL(yL, yR, src, dst, e1h, zh)
    return jnp.concatenate([outL[:N_NODES], outR[:N_NODES]], axis=1)
